# Initial kernel scaffold; baseline (speedup 1.0000x reference)
import jax
import jax.numpy as jnp
from jax.experimental import pallas as pl


def kernel(input, rows, cols, segment_ids):
    raise NotImplementedError("write your pallas kernel here")


# speedup vs baseline: 45.0686x; 45.0686x over previous
import jax
import jax.numpy as jnp
from jax import lax
from jax.experimental import pallas as pl
from jax.experimental.pallas import tpu as pltpu
from jax.experimental.pallas import tpu_sc as plsc

BS = 16
H = 512
W = 512
PATHS = 500
NSEG = BS * PATHS
NSEG_PAD = 8192
SEG_SL = NSEG_PAD // 16
TOT = 524288
NC = 2
NS = 16
L = 16
NW = NC * NS
P = TOT // NW
VEC = P // L
GCH = 128
NG = P // GCH


def _phase1(rows, cols, segs, flat, vals_o, sums_o, cnts_o,
            r_v, c_v, s_v, idx_v, vals_v, sums_v, cnts_v, tmp_v, red_v,
            sh_sums, sh_cnts, sem):
    cid = lax.axis_index("c")
    sid = lax.axis_index("s")
    wid = cid * NS + sid
    base = pl.multiple_of(wid * P, P)

    pltpu.sync_copy(rows.at[pl.ds(base, P)], r_v)
    pltpu.sync_copy(cols.at[pl.ds(base, P)], c_v)
    pltpu.sync_copy(segs.at[pl.ds(base, P)], s_v)

    zeros16 = jnp.zeros((L,), jnp.float32)
    ones16 = jnp.ones((L,), jnp.float32)

    def zbody(j, _):
        sl = pl.ds(j * L, L)
        sums_v[sl] = zeros16
        cnts_v[sl] = zeros16
        return 0
    lax.fori_loop(0, NSEG_PAD // L, zbody, 0)

    def ibody(j, _):
        sl = pl.ds(j * L, L)
        s = s_v[sl]
        b = (s * 8389) >> 22
        idx_v[sl] = b * (H * W) + r_v[sl] * W + c_v[sl]
        return 0
    lax.fori_loop(0, VEC, ibody, 0)

    def gbody(g, _):
        gsl = pl.ds(g * GCH, GCH)
        pltpu.async_copy(flat.at[idx_v.at[gsl]], vals_v.at[gsl], sem)
        return 0
    lax.fori_loop(0, NG, gbody, 0)
    pltpu.make_async_copy(flat.at[pl.ds(0, P)], vals_v, sem).wait()

    def abody(j, _):
        sl = pl.ds(j * L, L)
        s = s_v[sl]
        plsc.addupdate_scatter(sums_v, [s], vals_v[sl])
        plsc.addupdate_scatter(cnts_v, [s], ones16)
        return 0
    lax.fori_loop(0, VEC, abody, 0)

    pltpu.sync_copy(vals_v, vals_o.at[pl.ds(base, P)])

    pltpu.sync_copy(sums_v, sh_sums.at[sid])
    pltpu.sync_copy(cnts_v, sh_cnts.at[sid])
    plsc.subcore_barrier()

    off = pl.multiple_of(sid * SEG_SL, SEG_SL)
    for sh, out in ((sh_sums, sums_o), (sh_cnts, cnts_o)):
        pltpu.sync_copy(sh.at[0, pl.ds(off, SEG_SL)], red_v)
        for r0 in range(1, NS):
            pltpu.sync_copy(sh.at[r0, pl.ds(off, SEG_SL)], tmp_v)

            def rbody(j, _):
                sl = pl.ds(j * L, L)
                red_v[sl] = red_v[sl] + tmp_v[sl]
                return 0
            lax.fori_loop(0, SEG_SL // L, rbody, 0)
        pltpu.sync_copy(red_v, out.at[cid, sid])


def _phase2(vals, segs, sums, cnts, part_o,
            s_v, v_v, ps_v, pc_v, means_v, rc_v, acc_v, sem):
    cid = lax.axis_index("c")
    sid = lax.axis_index("s")
    wid = cid * NS + sid
    base = pl.multiple_of(wid * P, P)

    pltpu.sync_copy(vals.at[pl.ds(base, P)], v_v)
    pltpu.sync_copy(segs.at[pl.ds(base, P)], s_v)
    pltpu.sync_copy(sums, ps_v)
    pltpu.sync_copy(cnts, pc_v)

    one = jnp.float32(1.0)
    for row in range(NS):
        def mbody(j, _):
            sl = pl.ds(j * L, L)
            cnt = pc_v[0, row, sl] + pc_v[1, row, sl]
            safe = jnp.maximum(cnt, one)
            dst = pl.ds(row * SEG_SL + j * L, L)
            means_v[dst] = (ps_v[0, row, sl] + ps_v[1, row, sl]) / safe
            rc_v[dst] = one / safe
            return 0
        lax.fori_loop(0, SEG_SL // L, mbody, 0)

    def dbody(j, acc):
        sl = pl.ds(j * L, L)
        s = s_v[sl]
        m = plsc.load_gather(means_v, [s])
        rc = plsc.load_gather(rc_v, [s])
        return acc + jnp.abs(m - v_v[sl]) * rc
    acc = lax.fori_loop(0, VEC, dbody, jnp.zeros((L,), jnp.float32))
    acc_v[...] = acc
    pltpu.sync_copy(acc_v, part_o.at[wid])


def kernel(input, rows, cols, segment_ids):
    flat = input.reshape(-1)
    mesh = plsc.VectorSubcoreMesh(core_axis_name="c", subcore_axis_name="s")

    k1 = pl.kernel(
        _phase1,
        compiler_params=pltpu.CompilerParams(needs_layout_passes=False),
        out_type=(
            jax.ShapeDtypeStruct((TOT,), jnp.float32),
            jax.ShapeDtypeStruct((NC, NS, SEG_SL), jnp.float32),
            jax.ShapeDtypeStruct((NC, NS, SEG_SL), jnp.float32),
        ),
        mesh=mesh,
        scratch_types=[
            pltpu.VMEM((P,), jnp.int32),
            pltpu.VMEM((P,), jnp.int32),
            pltpu.VMEM((P,), jnp.int32),
            pltpu.VMEM((P,), jnp.int32),
            pltpu.VMEM((P,), jnp.float32),
            pltpu.VMEM((NSEG_PAD,), jnp.float32),
            pltpu.VMEM((NSEG_PAD,), jnp.float32),
            pltpu.VMEM((SEG_SL,), jnp.float32),
            pltpu.VMEM((SEG_SL,), jnp.float32),
            pltpu.VMEM_SHARED((NS, NSEG_PAD), jnp.float32),
            pltpu.VMEM_SHARED((NS, NSEG_PAD), jnp.float32),
            pltpu.SemaphoreType.DMA,
        ],
    )
    vals, sums, cnts = k1(rows, cols, segment_ids, flat)

    k2 = pl.kernel(
        _phase2,
        compiler_params=pltpu.CompilerParams(needs_layout_passes=False),
        out_type=jax.ShapeDtypeStruct((NW, L), jnp.float32),
        mesh=mesh,
        scratch_types=[
            pltpu.VMEM((P,), jnp.int32),
            pltpu.VMEM((P,), jnp.float32),
            pltpu.VMEM((NC, NS, SEG_SL), jnp.float32),
            pltpu.VMEM((NC, NS, SEG_SL), jnp.float32),
            pltpu.VMEM((NSEG_PAD,), jnp.float32),
            pltpu.VMEM((NSEG_PAD,), jnp.float32),
            pltpu.VMEM((L,), jnp.float32),
            pltpu.SemaphoreType.DMA,
        ],
    )
    part = k2(vals, segment_ids, sums, cnts)
    return jnp.sum(part) / jnp.float32(BS)
